# baseline (speedup 1.0000x reference)
import functools

import jax
import jax.numpy as jnp
from jax import lax
from jax.experimental import pallas as pl
from jax.experimental.pallas import tpu as pltpu
from jax.experimental.pallas import tpu_sc as plsc

N = 10000
E = 320000
NP = 10240
NC, NS = 2, 16
NW = NC * NS
NCH, CH = 80, 128
EPT = NCH * CH
E_PAD = NW * EPT
RPS = NP // NS

_MESH = plsc.VectorSubcoreMesh(
    core_axis_name="c", subcore_axis_name="s", num_cores=NC, num_subcores=NS
)


def _deg_body(dst_hbm, ones_hbm, z1_hbm, degp_hbm, dst_v, ones_v, acc):
    c = lax.axis_index("c")
    s = lax.axis_index("s")
    wid = s * NC + c
    r0 = s * RPS
    pltpu.sync_copy(z1_hbm.at[pl.ds(r0, RPS)], acc.at[pl.ds(r0, RPS)])
    pltpu.sync_copy(dst_hbm.at[wid], dst_v)
    pltpu.sync_copy(ones_hbm, ones_v)
    plsc.subcore_barrier()

    def step(j, carry):
        pltpu.sync_copy(ones_v, acc.at[dst_v.at[j]], add=True)
        return carry

    lax.fori_loop(0, NCH, step, 0)
    plsc.subcore_barrier()
    pltpu.sync_copy(acc.at[pl.ds(r0, RPS)], degp_hbm.at[c, pl.ds(r0, RPS)])


_deg_kernel = functools.partial(
    pl.kernel,
    _deg_body,
    out_type=jax.ShapeDtypeStruct((NC, NP), jnp.float32),
    mesh=_MESH,
    scratch_types=[
        pltpu.VMEM((NCH, CH), jnp.int32),
        pltpu.VMEM((CH,), jnp.float32),
        pltpu.VMEM_SHARED((NP,), jnp.float32),
    ],
)()


def _make_agg(num_slabs):
    def body(*refs):
        tables = refs[:num_slabs]
        edges_hbm, z2_hbm = refs[num_slabs:num_slabs + 2]
        outs = refs[num_slabs + 2:2 * num_slabs + 2]
        (idx0, idx1, rows0, rows1, isem0, isem1, gsem0, gsem1, acc) = \
            refs[2 * num_slabs + 2:]
        idxs = (idx0, idx1)
        rows = (rows0, rows1)
        isems = (isem0, isem1)
        gsems = (gsem0, gsem1)

        c = lax.axis_index("c")
        s = lax.axis_index("s")
        wid = s * NC + c
        r0 = s * RPS

        def idx_fetch(j, b):
            pltpu.async_copy(edges_hbm.at[wid, j], idxs[b], isems[b])

        def gather(table, j, b):
            pltpu.make_async_copy(edges_hbm.at[wid, j], idxs[b],
                                  isems[b]).wait()
            pltpu.async_copy(table.at[idxs[b].at[0]], rows[b], gsems[b])

        for h in range(num_slabs):
            table = tables[h]
            pltpu.sync_copy(z2_hbm.at[pl.ds(r0, RPS)], acc.at[pl.ds(r0, RPS)])
            idx_fetch(0, 0)
            idx_fetch(1, 1)
            gather(table, 0, 0)
            plsc.subcore_barrier()

            def group(g, carry, table=table):
                for b in range(2):
                    j = g * 2 + b

                    @pl.when(j + 1 < NCH)
                    def _():
                        gather(table, j + 1, 1 - b)

                    pltpu.make_async_copy(table.at[idxs[b].at[0]], rows[b],
                                          gsems[b]).wait()
                    pltpu.sync_copy(rows[b], acc.at[idxs[b].at[1]], add=True)

                    @pl.when(j + 2 < NCH)
                    def _():
                        idx_fetch(j + 2, b)
                return carry

            lax.fori_loop(0, NCH // 2, group, 0)
            plsc.subcore_barrier()
            pltpu.sync_copy(acc.at[pl.ds(r0, RPS)],
                            outs[h].at[c, pl.ds(r0, RPS)])
            if h + 1 < num_slabs:
                plsc.subcore_barrier()

    return functools.partial(
        pl.kernel,
        body,
        out_type=[jax.ShapeDtypeStruct((NC, NP, 128), jnp.float32)] * num_slabs,
        mesh=_MESH,
        scratch_types=[
            pltpu.VMEM((2, CH), jnp.int32),
            pltpu.VMEM((2, CH), jnp.int32),
            pltpu.VMEM((CH, 128), jnp.float32),
            pltpu.VMEM((CH, 128), jnp.float32),
            pltpu.SemaphoreType.DMA,
            pltpu.SemaphoreType.DMA,
            pltpu.SemaphoreType.DMA,
            pltpu.SemaphoreType.DMA,
            pltpu.VMEM_SHARED((NP, 128), jnp.float32),
        ],
    )()


_agg1 = _make_agg(1)
_agg2 = _make_agg(2)


_BM = 1280
_GRID = NP // _BM


def _mm1_body(x_ref, w_ref, o_ref):
    o_ref[...] = jnp.dot(x_ref[...], w_ref[...],
                         preferred_element_type=jnp.float32)


def _dinv_body(d0_ref, d1_ref, o_ref):
    deg = d0_ref[...] + d1_ref[...] + 1.0
    dinv = lax.rsqrt(jnp.maximum(deg, 1.0))
    o_ref[...] = jnp.broadcast_to(dinv, (_BM, 128))


def _scale_body(t1_ref, dinv_ref, a_ref, b_ref):
    dinv = dinv_ref[...]
    a_ref[...] = t1_ref[:, :128] * dinv
    b_ref[...] = t1_ref[:, 128:] * dinv


def _layer2_body(pa_ref, pb_ref, sa_ref, sb_ref, dinv_ref, b1_ref, w2_ref,
                 o_ref):
    dinv = dinv_ref[...]
    ha = jax.nn.relu(dinv * (pa_ref[0] + pa_ref[1] + sa_ref[...])
                     + b1_ref[0:1, :128])
    hb = jax.nn.relu(dinv * (pb_ref[0] + pb_ref[1] + sb_ref[...])
                     + b1_ref[0:1, 128:])
    t2 = (jnp.dot(ha, w2_ref[:128, :], preferred_element_type=jnp.float32)
          + jnp.dot(hb, w2_ref[128:, :], preferred_element_type=jnp.float32))
    o_ref[...] = t2 * dinv


def _final_body(p_ref, s_ref, dinv_ref, b2_ref, o_ref):
    o_ref[...] = (dinv_ref[...] * (p_ref[0] + p_ref[1] + s_ref[...])
                  + b2_ref[0:1, :])


def _row_spec(cols):
    return pl.BlockSpec((_BM, cols), lambda m: (m, 0))


def _whole_spec(shape):
    return pl.BlockSpec(shape, lambda m: tuple(0 for _ in shape))


def _part_spec():
    return pl.BlockSpec((NC, _BM, 128), lambda m: (0, m, 0))


def kernel(x, edge_index, W1, b1, W2, b2):
    pad = jnp.full((E_PAD - E,), NP - 1, jnp.int32)
    src_p = jnp.concatenate([edge_index[0], pad]).reshape(NW, NCH, CH)
    dst_p = jnp.concatenate([edge_index[1], pad]).reshape(NW, NCH, CH)
    edges = jnp.stack([src_p, dst_p], axis=2)
    ones_ch = jnp.ones((CH,), jnp.float32)
    z1 = jnp.zeros((NP,), jnp.float32)
    z2 = jnp.zeros((NP, 128), jnp.float32)
    x_pad = jnp.pad(x, ((0, NP - N), (0, 0)))

    degp = _deg_kernel(dst_p, ones_ch, z1)

    t1 = pl.pallas_call(
        _mm1_body,
        grid=(_GRID,),
        in_specs=[_row_spec(128), _whole_spec((128, 256))],
        out_specs=_row_spec(256),
        out_shape=jax.ShapeDtypeStruct((NP, 256), jnp.float32),
    )(x_pad, W1)

    dinv2d = pl.pallas_call(
        _dinv_body,
        grid=(_GRID,),
        in_specs=[_row_spec(1), _row_spec(1)],
        out_specs=_row_spec(128),
        out_shape=jax.ShapeDtypeStruct((NP, 128), jnp.float32),
    )(degp[0].reshape(NP, 1), degp[1].reshape(NP, 1))

    s1a, s1b = pl.pallas_call(
        _scale_body,
        grid=(_GRID,),
        in_specs=[_row_spec(256), _row_spec(128)],
        out_specs=[_row_spec(128), _row_spec(128)],
        out_shape=[jax.ShapeDtypeStruct((NP, 128), jnp.float32)] * 2,
    )(t1, dinv2d)

    p1a, p1b = _agg2(s1a, s1b, edges, z2)

    s2 = pl.pallas_call(
        _layer2_body,
        grid=(_GRID,),
        in_specs=[_part_spec(), _part_spec(), _row_spec(128), _row_spec(128),
                  _row_spec(128), _whole_spec((1, 256)),
                  _whole_spec((256, 128))],
        out_specs=_row_spec(128),
        out_shape=jax.ShapeDtypeStruct((NP, 128), jnp.float32),
    )(p1a, p1b, s1a, s1b, dinv2d, b1.reshape(1, 256), W2)

    (p2,) = _agg1(s2, edges, z2)

    out = pl.pallas_call(
        _final_body,
        grid=(_GRID,),
        in_specs=[_part_spec(), _row_spec(128), _row_spec(128),
                  _whole_spec((1, 128))],
        out_specs=_row_spec(128),
        out_shape=jax.ShapeDtypeStruct((NP, 128), jnp.float32),
    )(p2, s2, dinv2d, b2.reshape(1, 128))

    return out[:N]


# speedup vs baseline: 3.4167x; 3.4167x over previous
import functools

import jax
import jax.numpy as jnp
from jax import lax
from jax.experimental import pallas as pl
from jax.experimental.pallas import tpu as pltpu
from jax.experimental.pallas import tpu_sc as plsc

N = 10000
E = 320000
NP = 10240
NC, NS = 2, 16
NW = NC * NS
NCH, CH = 80, 128
EPT = NCH * CH
E_PAD = NW * EPT
RPS = NP // NS

_MESH = plsc.VectorSubcoreMesh(
    core_axis_name="c", subcore_axis_name="s", num_cores=NC, num_subcores=NS
)


def _deg_body(dst_hbm, ones_hbm, z1_hbm, degp_hbm, dst_v, ones_v, acc):
    c = lax.axis_index("c")
    s = lax.axis_index("s")
    wid = s * NC + c
    r0 = s * RPS
    pltpu.sync_copy(z1_hbm.at[pl.ds(r0, RPS)], acc.at[pl.ds(r0, RPS)])
    pltpu.sync_copy(dst_hbm.at[wid], dst_v)
    pltpu.sync_copy(ones_hbm, ones_v)
    plsc.subcore_barrier()

    def step(j, carry):
        pltpu.sync_copy(ones_v, acc.at[dst_v.at[j]], add=True)
        return carry

    lax.fori_loop(0, NCH, step, 0)
    plsc.subcore_barrier()
    pltpu.sync_copy(acc.at[pl.ds(r0, RPS)], degp_hbm.at[c, pl.ds(r0, RPS)])


_deg_kernel = functools.partial(
    pl.kernel,
    _deg_body,
    out_type=jax.ShapeDtypeStruct((NC, NP), jnp.float32),
    mesh=_MESH,
    scratch_types=[
        pltpu.VMEM((NCH, CH), jnp.int32),
        pltpu.VMEM((CH,), jnp.float32),
        pltpu.VMEM_SHARED((NP,), jnp.float32),
    ],
)()


def _make_agg(num_slabs):
    def body(*refs):
        tables = refs[:num_slabs]
        edges_hbm, z2_hbm = refs[num_slabs:num_slabs + 2]
        outs = refs[num_slabs + 2:2 * num_slabs + 2]
        (idx0, idx1, rows0, rows1, isem0, isem1, gsem0, gsem1, acc) = \
            refs[2 * num_slabs + 2:]
        idxs = (idx0, idx1)
        rows = (rows0, rows1)
        isems = (isem0, isem1)
        gsems = (gsem0, gsem1)

        c = lax.axis_index("c")
        s = lax.axis_index("s")
        wid = s * NC + c
        r0 = s * RPS

        def idx_fetch(j, b):
            pltpu.async_copy(edges_hbm.at[wid, j], idxs[b], isems[b])

        def gather(table, j, b):
            pltpu.make_async_copy(edges_hbm.at[wid, j], idxs[b],
                                  isems[b]).wait()
            pltpu.async_copy(table.at[idxs[b].at[0]], rows[b], gsems[b])

        for h in range(num_slabs):
            table = tables[h]
            pltpu.sync_copy(z2_hbm.at[pl.ds(r0, RPS)], acc.at[pl.ds(r0, RPS)])
            idx_fetch(0, 0)
            idx_fetch(1, 1)
            gather(table, 0, 0)
            plsc.subcore_barrier()

            def group(g, carry, table=table):
                for b in range(2):
                    j = g * 2 + b

                    @pl.when(j + 1 < NCH)
                    def _():
                        gather(table, j + 1, 1 - b)

                    pltpu.make_async_copy(table.at[idxs[b].at[0]], rows[b],
                                          gsems[b]).wait()
                    pltpu.sync_copy(rows[b], acc.at[idxs[b].at[1]], add=True)

                    @pl.when(j + 2 < NCH)
                    def _():
                        idx_fetch(j + 2, b)
                return carry

            lax.fori_loop(0, NCH // 2, group, 0)
            plsc.subcore_barrier()
            pltpu.sync_copy(acc.at[pl.ds(r0, RPS)],
                            outs[h].at[c, pl.ds(r0, RPS)])
            if h + 1 < num_slabs:
                plsc.subcore_barrier()

    return functools.partial(
        pl.kernel,
        body,
        out_type=[jax.ShapeDtypeStruct((NC, NP, 128), jnp.float32)] * num_slabs,
        mesh=_MESH,
        scratch_types=[
            pltpu.VMEM((2, CH), jnp.int32),
            pltpu.VMEM((2, CH), jnp.int32),
            pltpu.VMEM((CH, 128), jnp.float32),
            pltpu.VMEM((CH, 128), jnp.float32),
            pltpu.SemaphoreType.DMA,
            pltpu.SemaphoreType.DMA,
            pltpu.SemaphoreType.DMA,
            pltpu.SemaphoreType.DMA,
            pltpu.VMEM_SHARED((NP, 128), jnp.float32),
        ],
    )()


_agg1 = _make_agg(1)
_agg2 = _make_agg(2)


_BM = 1280
_GRID = NP // _BM


def _mm1_body(x_ref, w_ref, o_ref):
    o_ref[...] = jnp.dot(x_ref[...], w_ref[...],
                         preferred_element_type=jnp.float32)


def _dinv_body(d0_ref, d1_ref, o_ref):
    deg = d0_ref[...] + d1_ref[...] + 1.0
    dinv = lax.rsqrt(jnp.maximum(deg, 1.0))
    o_ref[...] = jnp.broadcast_to(dinv, (_BM, 128))


def _scale_body(t1_ref, dinv_ref, a_ref, b_ref):
    dinv = dinv_ref[...]
    a_ref[...] = t1_ref[:, :128] * dinv
    b_ref[...] = t1_ref[:, 128:] * dinv


def _layer2_body(pa_ref, pb_ref, sa_ref, sb_ref, dinv_ref, b1_ref, w2_ref,
                 o_ref):
    dinv = dinv_ref[...]
    ha = jax.nn.relu(dinv * (pa_ref[0] + pa_ref[1] + sa_ref[...])
                     + b1_ref[0:1, :128])
    hb = jax.nn.relu(dinv * (pb_ref[0] + pb_ref[1] + sb_ref[...])
                     + b1_ref[0:1, 128:])
    t2 = (jnp.dot(ha, w2_ref[:128, :], preferred_element_type=jnp.float32)
          + jnp.dot(hb, w2_ref[128:, :], preferred_element_type=jnp.float32))
    o_ref[...] = t2 * dinv


def _final_body(p_ref, s_ref, dinv_ref, b2_ref, o_ref):
    o_ref[...] = (dinv_ref[...] * (p_ref[0] + p_ref[1] + s_ref[...])
                  + b2_ref[0:1, :])


def _row_spec(cols):
    return pl.BlockSpec((_BM, cols), lambda m: (m, 0))


def _whole_spec(shape):
    return pl.BlockSpec(shape, lambda m: tuple(0 for _ in shape))


def _part_spec():
    return pl.BlockSpec((NC, _BM, 128), lambda m: (0, m, 0))


def kernel(x, edge_index, W1, b1, W2, b2):
    pad = (jnp.arange(E_PAD - E, dtype=jnp.int32) % (NP - N)) + N
    src_p = jnp.concatenate([edge_index[0], pad]).reshape(NW, NCH, CH)
    dst_p = jnp.concatenate([edge_index[1], pad]).reshape(NW, NCH, CH)
    edges = jnp.stack([src_p, dst_p], axis=2)
    ones_ch = jnp.ones((CH,), jnp.float32)
    z1 = jnp.zeros((NP,), jnp.float32)
    z2 = jnp.zeros((NP, 128), jnp.float32)
    x_pad = jnp.pad(x, ((0, NP - N), (0, 0)))

    degp = _deg_kernel(dst_p, ones_ch, z1)

    t1 = pl.pallas_call(
        _mm1_body,
        grid=(_GRID,),
        in_specs=[_row_spec(128), _whole_spec((128, 256))],
        out_specs=_row_spec(256),
        out_shape=jax.ShapeDtypeStruct((NP, 256), jnp.float32),
    )(x_pad, W1)

    dinv2d = pl.pallas_call(
        _dinv_body,
        grid=(_GRID,),
        in_specs=[_row_spec(1), _row_spec(1)],
        out_specs=_row_spec(128),
        out_shape=jax.ShapeDtypeStruct((NP, 128), jnp.float32),
    )(degp[0].reshape(NP, 1), degp[1].reshape(NP, 1))

    s1a, s1b = pl.pallas_call(
        _scale_body,
        grid=(_GRID,),
        in_specs=[_row_spec(256), _row_spec(128)],
        out_specs=[_row_spec(128), _row_spec(128)],
        out_shape=[jax.ShapeDtypeStruct((NP, 128), jnp.float32)] * 2,
    )(t1, dinv2d)

    p1a, p1b = _agg2(s1a, s1b, edges, z2)

    s2 = pl.pallas_call(
        _layer2_body,
        grid=(_GRID,),
        in_specs=[_part_spec(), _part_spec(), _row_spec(128), _row_spec(128),
                  _row_spec(128), _whole_spec((1, 256)),
                  _whole_spec((256, 128))],
        out_specs=_row_spec(128),
        out_shape=jax.ShapeDtypeStruct((NP, 128), jnp.float32),
    )(p1a, p1b, s1a, s1b, dinv2d, b1.reshape(1, 256), W2)

    (p2,) = _agg1(s2, edges, z2)

    out = pl.pallas_call(
        _final_body,
        grid=(_GRID,),
        in_specs=[_part_spec(), _row_spec(128), _row_spec(128),
                  _whole_spec((1, 128))],
        out_specs=_row_spec(128),
        out_shape=jax.ShapeDtypeStruct((NP, 128), jnp.float32),
    )(p2, s2, dinv2d, b2.reshape(1, 128))

    return out[:N]
